# baseline (speedup 1.0000x reference)
import jax
import jax.numpy as jnp
from jax.experimental import pallas as pl
from jax.experimental.pallas import tpu as pltpu


_G = 2


def _body(idx_ref, k_ref, v_ref, xk_ref, xv_ref, ok_ref, ov_ref):
    idx = idx_ref[0]
    ok_ref[...] = k_ref[...]
    ov_ref[...] = v_ref[...]
    ok_ref[:, pl.ds(idx, 1), :] = xk_ref[...]
    ov_ref[:, pl.ds(idx, 1), :] = xv_ref[...]


def kernel(k_cache, v_cache, xk, xv, layer_past_len):
    B, Hkv, S, D = k_cache.shape
    N = B * Hkv
    idx = jnp.asarray(layer_past_len, jnp.int32).reshape((1,))
    k3 = k_cache.reshape(N, S, D)
    v3 = v_cache.reshape(N, S, D)
    xk3 = xk.reshape(N, 1, D)
    xv3 = xv.reshape(N, 1, D)
    cache_spec = pl.BlockSpec((_G, S, D), lambda i, idx_ref: (i, 0, 0))
    x_spec = pl.BlockSpec((_G, 1, D), lambda i, idx_ref: (i, 0, 0))
    grid_spec = pltpu.PrefetchScalarGridSpec(
        num_scalar_prefetch=1,
        grid=(N // _G,),
        in_specs=[cache_spec, cache_spec, x_spec, x_spec],
        out_specs=[cache_spec, cache_spec],
    )
    ok, ov = pl.pallas_call(
        _body,
        grid_spec=grid_spec,
        out_shape=(
            jax.ShapeDtypeStruct(k3.shape, k3.dtype),
            jax.ShapeDtypeStruct(v3.shape, v3.dtype),
        ),
    )(idx, k3, v3, xk3, xv3)
    return ok.reshape(B, Hkv, S, D), ov.reshape(B, Hkv, S, D)


# speedup vs baseline: 1.0165x; 1.0165x over previous
import jax
import jax.numpy as jnp
from jax.experimental import pallas as pl
from jax.experimental.pallas import tpu as pltpu


_G = 8


def _body(idx_ref, c_ref, x_ref, o_ref):
    idx = idx_ref[0]
    o_ref[...] = c_ref[...]
    o_ref[:, pl.ds(idx, 1), :] = x_ref[...]


def _update_one(cache3, x3, idx):
    N, S, D = cache3.shape
    cache_spec = pl.BlockSpec((_G, S, D), lambda i, idx_ref: (i, 0, 0))
    x_spec = pl.BlockSpec((_G, 1, D), lambda i, idx_ref: (i, 0, 0))
    grid_spec = pltpu.PrefetchScalarGridSpec(
        num_scalar_prefetch=1,
        grid=(N // _G,),
        in_specs=[cache_spec, x_spec],
        out_specs=cache_spec,
    )
    return pl.pallas_call(
        _body,
        grid_spec=grid_spec,
        out_shape=jax.ShapeDtypeStruct(cache3.shape, cache3.dtype),
    )(idx, cache3, x3)


def kernel(k_cache, v_cache, xk, xv, layer_past_len):
    B, Hkv, S, D = k_cache.shape
    N = B * Hkv
    idx = jnp.asarray(layer_past_len, jnp.int32).reshape((1,))
    ok = _update_one(k_cache.reshape(N, S, D), xk.reshape(N, 1, D), idx)
    ov = _update_one(v_cache.reshape(N, S, D), xv.reshape(N, 1, D), idx)
    return ok.reshape(B, Hkv, S, D), ov.reshape(B, Hkv, S, D)
